# baseline (speedup 1.0000x reference)
import functools

import jax
import jax.numpy as jnp
from jax import lax
from jax.experimental import pallas as pl
from jax.experimental.pallas import tpu as pltpu
from jax.experimental.pallas import tpu_sc as plsc

NC, NS = 2, 16
NW = NC * NS
LANES = 16
GROWS = 128
NBUF = 4


def _sc_gather_sums(story2d, cm_i, *, d3, M, n_slots):
    d3w = d3 // 2
    slots_w = n_slots // NW
    rows_w = slots_w * M // GROWS
    spg = GROWS // M
    n_sec = 10
    gps = rows_w // n_sec
    sec_slots = slots_w // n_sec
    mesh = plsc.VectorSubcoreMesh(
        core_axis_name="c", subcore_axis_name="s",
        num_cores=NC, num_subcores=NS)
    mask = jnp.int32(-65536)

    d = d3 // 3
    @functools.partial(
        pl.kernel,
        out_type=jax.ShapeDtypeStruct((3, n_slots, d), jnp.float32),
        mesh=mesh,
        scratch_types=[
            pltpu.VMEM((rows_w, GROWS), jnp.int32),
            pltpu.VMEM((NBUF, GROWS, d3w), jnp.int32),
            [pltpu.VMEM((sec_slots, d), jnp.float32)] * 3,
            [pltpu.SemaphoreType.DMA] * NBUF,
        ],
        compiler_params=pltpu.CompilerParams(use_tc_tiling_on_sc=False,
                                             needs_layout_passes=False),
    )
    def k(story_ref, cm_ref, m_ref, idx_v, rows_v, outs, sems):
        ng = d3w // LANES
        gpd = d // LANES
        dest = []
        for g in range(ng):
            if g < gpd:
                dest.append(((outs[0], g * LANES), (outs[1], g * LANES)))
            else:
                j = (g - gpd) * LANES
                dest.append(((outs[2], j), (outs[2], d // 2 + j)))
        wid = lax.axis_index("s") * NC + lax.axis_index("c")
        pltpu.sync_copy(story_ref.at[pl.ds(wid * rows_w, rows_w)], idx_v)

        def fire(row, p):
            pltpu.async_copy(cm_ref.at[idx_v.at[row]], rows_v.at[p], sems[p])

        def drain(p):
            pltpu.make_async_copy(cm_ref.at[idx_v.at[0]], rows_v.at[p],
                                  sems[p]).wait()

        def compute(p, q):
            def slot(s8, cc):
                for jj in range(ng):
                    sl = pl.ds(jj * LANES, LANES)
                    xi = rows_v[p, s8 * M, sl]
                    lo = plsc.bitcast(xi << 16, jnp.float32)
                    hi = plsc.bitcast(xi & mask, jnp.float32)
                    for mm in range(1, M):
                        xi = rows_v[p, s8 * M + mm, sl]
                        lo = lo + plsc.bitcast(xi << 16, jnp.float32)
                        hi = hi + plsc.bitcast(xi & mask, jnp.float32)
                    (lo_ref, lo_c), (hi_ref, hi_c) = dest[jj]
                    lo_ref[q * spg + s8, pl.ds(lo_c, LANES)] = lo
                    hi_ref[q * spg + s8, pl.ds(hi_c, LANES)] = hi
                return cc
            lax.fori_loop(0, spg, slot, 0)

        def section(h, c):
            base = h * gps
            for p in range(NBUF):
                fire(base + p, p)

            def quad(j, cc):
                q0 = NBUF * j
                for p in range(NBUF):
                    drain(p)
                    compute(p, q0 + p)

                    @pl.when(j < gps // NBUF - 1)
                    def _():
                        fire(base + q0 + p + NBUF, p)
                return cc
            lax.fori_loop(0, gps // NBUF, quad, 0)
            for t in range(3):
                pltpu.sync_copy(
                    outs[t],
                    m_ref.at[t, pl.ds(wid * slots_w + h * sec_slots,
                                      sec_slots)])
            return c
        lax.fori_loop(0, n_sec, section, 0)

    return k(story2d, cm_i)


def _tc_attention(m, *, B, S, d, BB=128):
    inv_s = 1.0 / S

    def body(m1_ref, m2_ref, m3_ref, u_ref):
        m1 = m1_ref[0].reshape(BB, S, d)
        m2 = m2_ref[0].reshape(BB, S, d)
        m3 = m3_ref[0].reshape(BB, S, d)
        u1 = jnp.sum(m1, axis=1) * inv_s
        p1 = jax.nn.softmax(jnp.sum(m1 * u1[:, None, :], axis=2), axis=1)
        u2 = u1 + jnp.sum(m2 * p1[:, :, None], axis=1)
        p2 = jax.nn.softmax(jnp.sum(m2 * u2[:, None, :], axis=2), axis=1)
        u3 = u2 + jnp.sum(m3 * p2[:, :, None], axis=1)
        u_ref[...] = u3

    spec = lambda t: pl.BlockSpec((1, BB * S, d), lambda i, t=t: (t, i, 0))
    return pl.pallas_call(
        body,
        grid=(B // BB,),
        in_specs=[spec(0), spec(1), spec(2)],
        out_specs=pl.BlockSpec((BB, d), lambda i: (i, 0)),
        out_shape=jax.ShapeDtypeStruct((B, d), jnp.float32),
    )(m, m, m)


def kernel(story, C):
    S, B, M = story.shape
    V, d = C.shape[1], C.shape[2]
    n_slots = B * S
    st = jnp.transpose(story.astype(jnp.int32), (1, 0, 2))
    story2d = st.reshape(n_slots * M // GROWS, GROWS)
    tu = lax.bitcast_convert_type(C, jnp.uint32)
    r = ((tu + jnp.uint32(0x7FFF) + ((tu >> jnp.uint32(16)) & jnp.uint32(1)))
         & jnp.uint32(0xFFFF0000))
    c12 = (r[1] >> jnp.uint32(16)) | r[2]
    c3p = (r[3][:, :d // 2] >> jnp.uint32(16)) | r[3][:, d // 2:]
    cm_i = lax.bitcast_convert_type(
        jnp.concatenate([c12, c3p], axis=1), jnp.int32)
    m = _sc_gather_sums(story2d, cm_i, d3=3 * d, M=M, n_slots=n_slots)
    return _tc_attention(m, B=B, S=S, d=d)


# speedup vs baseline: 1.0028x; 1.0028x over previous
import functools

import jax
import jax.numpy as jnp
from jax import lax
from jax.experimental import pallas as pl
from jax.experimental.pallas import tpu as pltpu
from jax.experimental.pallas import tpu_sc as plsc

NC, NS = 2, 16
NW = NC * NS
LANES = 16
GROWS = 128
NBUF = 4


def _sc_gather_sums(story2d, cm_i, *, d3, M, n_slots):
    d3w = d3 // 2
    slots_w = n_slots // NW
    rows_w = slots_w * M // GROWS
    spg = GROWS // M
    n_sec = 10
    gps = rows_w // n_sec
    sec_slots = slots_w // n_sec
    mesh = plsc.VectorSubcoreMesh(
        core_axis_name="c", subcore_axis_name="s",
        num_cores=NC, num_subcores=NS)
    mask = jnp.int32(-65536)

    d = d3 // 3
    @functools.partial(
        pl.kernel,
        out_type=jax.ShapeDtypeStruct((3, n_slots, d), jnp.float32),
        mesh=mesh,
        scratch_types=[
            pltpu.VMEM((rows_w, GROWS), jnp.int32),
            pltpu.VMEM((NBUF, GROWS, d3w), jnp.int32),
            [pltpu.VMEM((sec_slots, d), jnp.float32)] * 3,
            [pltpu.SemaphoreType.DMA] * NBUF,
        ],
        compiler_params=pltpu.CompilerParams(use_tc_tiling_on_sc=False,
                                             needs_layout_passes=False),
    )
    def k(story_ref, cm_ref, m_ref, idx_v, rows_v, outs, sems):
        ng = d3w // LANES
        gpd = d // LANES
        dest = []
        for g in range(ng):
            if g < gpd:
                dest.append(((outs[0], g * LANES), (outs[1], g * LANES)))
            else:
                j = (g - gpd) * LANES
                dest.append(((outs[2], j), (outs[2], d // 2 + j)))
        wid = lax.axis_index("s") * NC + lax.axis_index("c")
        pltpu.sync_copy(story_ref.at[pl.ds(wid * rows_w, rows_w)], idx_v)

        def fire(row, p):
            pltpu.async_copy(cm_ref.at[idx_v.at[row]], rows_v.at[p], sems[p])

        def drain(p):
            pltpu.make_async_copy(cm_ref.at[idx_v.at[0]], rows_v.at[p],
                                  sems[p]).wait()

        def compute(p, q):
            def slot(s8, cc):
                for jj in range(ng):
                    sl = pl.ds(jj * LANES, LANES)
                    xi = rows_v[p, s8 * M, sl]
                    lo = plsc.bitcast(xi << 16, jnp.float32)
                    hi = plsc.bitcast(xi & mask, jnp.float32)
                    for mm in range(1, M):
                        xi = rows_v[p, s8 * M + mm, sl]
                        lo = lo + plsc.bitcast(xi << 16, jnp.float32)
                        hi = hi + plsc.bitcast(xi & mask, jnp.float32)
                    (lo_ref, lo_c), (hi_ref, hi_c) = dest[jj]
                    lo_ref[q * spg + s8, pl.ds(lo_c, LANES)] = lo
                    hi_ref[q * spg + s8, pl.ds(hi_c, LANES)] = hi
                return cc
            lax.fori_loop(0, spg, slot, 0)

        def section(h, c):
            base = h * gps
            for p in range(NBUF):
                fire(base + p, p)

            def quad(j, cc):
                q0 = NBUF * j
                for p in range(NBUF):
                    drain(p)
                    compute(p, q0 + p)

                    @pl.when(j < gps // NBUF - 1)
                    def _():
                        fire(base + q0 + p + NBUF, p)
                return cc
            lax.fori_loop(0, gps // NBUF, quad, 0)
            for t in range(3):
                pltpu.sync_copy(
                    outs[t],
                    m_ref.at[t, pl.ds(wid * slots_w + h * sec_slots,
                                      sec_slots)])
            return c
        lax.fori_loop(0, n_sec, section, 0)

    return k(story2d, cm_i)


def _tc_attention(m, *, B, S, d, BB=128):
    inv_s = 1.0 / S

    def body(m1_ref, m2_ref, m3_ref, u_ref):
        m1 = m1_ref[0].reshape(BB, S, d)
        m2 = m2_ref[0].reshape(BB, S, d)
        m3 = m3_ref[0].reshape(BB, S, d)
        u1 = jnp.sum(m1, axis=1) * inv_s
        p1 = jax.nn.softmax(jnp.sum(m1 * u1[:, None, :], axis=2), axis=1)
        u2 = u1 + jnp.sum(m2 * p1[:, :, None], axis=1)
        p2 = jax.nn.softmax(jnp.sum(m2 * u2[:, None, :], axis=2), axis=1)
        u3 = u2 + jnp.sum(m3 * p2[:, :, None], axis=1)
        u_ref[...] = u3

    spec = lambda t: pl.BlockSpec((1, BB * S, d), lambda i, t=t: (t, i, 0))
    return pl.pallas_call(
        body,
        grid=(B // BB,),
        in_specs=[spec(0), spec(1), spec(2)],
        out_specs=pl.BlockSpec((BB, d), lambda i: (i, 0)),
        out_shape=jax.ShapeDtypeStruct((B, d), jnp.float32),
    )(m, m, m)


def kernel(story, C):
    S, B, M = story.shape
    V, d = C.shape[1], C.shape[2]
    n_slots = B * S
    st = jnp.transpose(story.astype(jnp.int32), (1, 0, 2))
    story2d = st.reshape(n_slots * M // GROWS, GROWS)
    tu = lax.bitcast_convert_type(C, jnp.uint32)
    hm = jnp.uint32(0xFFFF0000)
    c12 = (tu[1] >> jnp.uint32(16)) | (tu[2] & hm)
    c3p = (tu[3][:, :d // 2] >> jnp.uint32(16)) | (tu[3][:, d // 2:] & hm)
    cm_i = lax.bitcast_convert_type(
        jnp.concatenate([c12, c3p], axis=1), jnp.int32)
    m = _sc_gather_sums(story2d, cm_i, d3=3 * d, M=M, n_slots=n_slots)
    return _tc_attention(m, B=B, S=S, d=d)


# speedup vs baseline: 1.0723x; 1.0693x over previous
import functools

import jax
import jax.numpy as jnp
import numpy as np
from jax import lax
from jax.experimental import pallas as pl
from jax.experimental.pallas import tpu as pltpu
from jax.experimental.pallas import tpu_sc as plsc

NC, NS = 2, 16
NW = NC * NS
LANES = 16
GROWS = 128
NBUF = 4


def _sc_gather_sums(story2d, cm_i, *, d3, M, n_slots):
    d3w = d3 // 2
    slots_w = n_slots // NW
    rows_w = slots_w * M // GROWS
    spg = GROWS // M
    n_sec = 10
    gps = rows_w // n_sec
    sec_slots = slots_w // n_sec
    mesh = plsc.VectorSubcoreMesh(
        core_axis_name="c", subcore_axis_name="s",
        num_cores=NC, num_subcores=NS)
    mask = jnp.int32(-65536)

    d = d3 // 3
    @functools.partial(
        pl.kernel,
        out_type=jax.ShapeDtypeStruct((3, n_slots, d), jnp.float32),
        mesh=mesh,
        scratch_types=[
            pltpu.VMEM((rows_w, GROWS), jnp.int32),
            pltpu.VMEM((NBUF, GROWS, d3w), jnp.int32),
            [pltpu.VMEM((sec_slots, d), jnp.float32)] * 3,
            [pltpu.SemaphoreType.DMA] * NBUF,
        ],
        compiler_params=pltpu.CompilerParams(use_tc_tiling_on_sc=False,
                                             needs_layout_passes=False),
    )
    def k(story_ref, cm_ref, m_ref, idx_v, rows_v, outs, sems):
        ng = d3w // LANES
        gpd = d // LANES
        dest = []
        for g in range(ng):
            if g < gpd:
                dest.append(((outs[0], g * LANES), (outs[1], g * LANES)))
            else:
                j = (g - gpd) * LANES
                dest.append(((outs[2], j), (outs[2], d // 2 + j)))
        wid = lax.axis_index("s") * NC + lax.axis_index("c")
        pltpu.sync_copy(story_ref.at[pl.ds(wid * rows_w, rows_w)], idx_v)

        def fire(row, p):
            pltpu.async_copy(cm_ref.at[idx_v.at[row]], rows_v.at[p], sems[p])

        def drain(p):
            pltpu.make_async_copy(cm_ref.at[idx_v.at[0]], rows_v.at[p],
                                  sems[p]).wait()

        def compute(p, q):
            def slot(s8, cc):
                for jj in range(ng):
                    sl = pl.ds(jj * LANES, LANES)
                    xi = rows_v[p, s8 * M, sl]
                    lo = plsc.bitcast(xi << 16, jnp.float32)
                    hi = plsc.bitcast(xi & mask, jnp.float32)
                    for mm in range(1, M):
                        xi = rows_v[p, s8 * M + mm, sl]
                        lo = lo + plsc.bitcast(xi << 16, jnp.float32)
                        hi = hi + plsc.bitcast(xi & mask, jnp.float32)
                    (lo_ref, lo_c), (hi_ref, hi_c) = dest[jj]
                    lo_ref[q * spg + s8, pl.ds(lo_c, LANES)] = lo
                    hi_ref[q * spg + s8, pl.ds(hi_c, LANES)] = hi
                return cc
            lax.fori_loop(0, spg, slot, 0)

        def section(h, c):
            base = h * gps
            for p in range(NBUF):
                fire(base + p, p)

            def quad(j, cc):
                q0 = NBUF * j
                for p in range(NBUF):
                    drain(p)
                    compute(p, q0 + p)

                    @pl.when(j < gps // NBUF - 1)
                    def _():
                        fire(base + q0 + p + NBUF, p)
                return cc
            lax.fori_loop(0, gps // NBUF, quad, 0)
            for t in range(3):
                pltpu.sync_copy(
                    outs[t],
                    m_ref.at[t, pl.ds(wid * slots_w + h * sec_slots,
                                      sec_slots)])
            return c
        lax.fori_loop(0, n_sec, section, 0)

    return k(story2d, cm_i)


def _tc_pack(tu, *, V, d, VB=2000):
    def body(a_ref, b_ref, c_ref, o_ref):
        h = np.uint32(0x7FFF)
        one = np.uint32(1)
        s16 = np.uint32(16)
        hm = np.uint32(0xFFFF0000)

        def rnd(x):
            return (x + h + ((x >> s16) & one)) & hm

        a = rnd(a_ref[0])
        b = rnd(b_ref[0])
        c = rnd(c_ref[0])
        o_ref[:, :d] = (a >> s16) | b
        o_ref[:, d:] = (c[:, :d // 2] >> s16) | c[:, d // 2:]

    spec = lambda t: pl.BlockSpec((1, VB, d), lambda i, t=t: (t, i, 0))
    out = pl.pallas_call(
        body,
        grid=(V // VB,),
        in_specs=[spec(1), spec(2), spec(3)],
        out_specs=pl.BlockSpec((VB, 3 * d // 2), lambda i: (i, 0)),
        out_shape=jax.ShapeDtypeStruct((V, 3 * d // 2), jnp.uint32),
    )(tu, tu, tu)
    return lax.bitcast_convert_type(out, jnp.int32)


def _tc_attention(m, *, B, S, d, BB=128):
    inv_s = 1.0 / S

    def body(m1_ref, m2_ref, m3_ref, u_ref):
        m1 = m1_ref[0].reshape(BB, S, d)
        m2 = m2_ref[0].reshape(BB, S, d)
        m3 = m3_ref[0].reshape(BB, S, d)
        u1 = jnp.sum(m1, axis=1) * inv_s
        p1 = jax.nn.softmax(jnp.sum(m1 * u1[:, None, :], axis=2), axis=1)
        u2 = u1 + jnp.sum(m2 * p1[:, :, None], axis=1)
        p2 = jax.nn.softmax(jnp.sum(m2 * u2[:, None, :], axis=2), axis=1)
        u3 = u2 + jnp.sum(m3 * p2[:, :, None], axis=1)
        u_ref[...] = u3

    spec = lambda t: pl.BlockSpec((1, BB * S, d), lambda i, t=t: (t, i, 0))
    return pl.pallas_call(
        body,
        grid=(B // BB,),
        in_specs=[spec(0), spec(1), spec(2)],
        out_specs=pl.BlockSpec((BB, d), lambda i: (i, 0)),
        out_shape=jax.ShapeDtypeStruct((B, d), jnp.float32),
    )(m, m, m)


def kernel(story, C):
    S, B, M = story.shape
    V, d = C.shape[1], C.shape[2]
    n_slots = B * S
    st = jnp.transpose(story.astype(jnp.int32), (1, 0, 2))
    story2d = st.reshape(n_slots * M // GROWS, GROWS)
    tu = lax.bitcast_convert_type(C, jnp.uint32)
    cm_i = _tc_pack(tu, V=V, d=d)
    m = _sc_gather_sums(story2d, cm_i, d3=3 * d, M=M, n_slots=n_slots)
    return _tc_attention(m, B=B, S=S, d=d)
